# Initial kernel scaffold; baseline (speedup 1.0000x reference)
import functools

import jax
import jax.numpy as jnp
from jax import lax
from jax.experimental import pallas as pl
from jax.experimental.pallas import tpu as pltpu
from jax.experimental.pallas import tpu_sc as plsc

_LANES = 16
_CHUNK = 512
_SUB = 128


def kernel(F0, F1, F2, indices_list):
    n = indices_list.shape[1]
    r = F0.shape[1]
    info = plsc.get_sparse_core_info()
    num_workers = info.num_cores * info.num_subcores
    n_per_w = n // num_workers
    n_chunks = n_per_w // _CHUNK

    mesh = plsc.VectorSubcoreMesh(core_axis_name="c", subcore_axis_name="s")

    @functools.partial(
        pl.kernel,
        out_type=jax.ShapeDtypeStruct((n,), jnp.float32),
        mesh=mesh,
        scratch_types=[
            pltpu.VMEM((_CHUNK,), jnp.int32),
            pltpu.VMEM((_CHUNK,), jnp.int32),
            pltpu.VMEM((_CHUNK,), jnp.int32),
            pltpu.VMEM((_CHUNK, r), jnp.float32),
            pltpu.VMEM((_CHUNK, r), jnp.float32),
            pltpu.VMEM((_CHUNK, r), jnp.float32),
            pltpu.VMEM((_CHUNK,), jnp.float32),
            pltpu.SemaphoreType.DMA,
        ],
    )
    def body(f0_hbm, f1_hbm, f2_hbm, idx_hbm, out_hbm,
             i0, i1, i2, g0, g1, g2, ov, sem):
        wid = lax.axis_index("s") * info.num_cores + lax.axis_index("c")
        wbase = wid * n_per_w
        iota = lax.iota(jnp.int32, _LANES)

        def do_chunk(c, carry):
            base = wbase + c * _CHUNK
            pltpu.sync_copy(idx_hbm.at[0, pl.ds(base, _CHUNK)], i0)
            pltpu.sync_copy(idx_hbm.at[1, pl.ds(base, _CHUNK)], i1)
            pltpu.sync_copy(idx_hbm.at[2, pl.ds(base, _CHUNK)], i2)
            copies = []
            for j in range(_CHUNK // _SUB):
                s = pl.ds(j * _SUB, _SUB)
                copies.append(pltpu.async_copy(f0_hbm.at[i0.at[s]], g0.at[s], sem))
                copies.append(pltpu.async_copy(f1_hbm.at[i1.at[s]], g1.at[s], sem))
                copies.append(pltpu.async_copy(f2_hbm.at[i2.at[s]], g2.at[s], sem))
            for cp in copies:
                cp.wait()

            def do_group(g, gcarry):
                rows = g * _LANES + iota
                acc = jnp.zeros((_LANES,), jnp.float32)
                for rr in range(r):
                    col = jnp.full((_LANES,), rr, jnp.int32)
                    v0 = plsc.load_gather(g0, [rows, col])
                    v1 = plsc.load_gather(g1, [rows, col])
                    v2 = plsc.load_gather(g2, [rows, col])
                    acc = acc + v0 * v1 * v2
                ov[pl.ds(g * _LANES, _LANES)] = acc
                return gcarry

            lax.fori_loop(0, _CHUNK // _LANES, do_group, 0)
            pltpu.sync_copy(ov, out_hbm.at[pl.ds(base, _CHUNK)])
            return carry

        lax.fori_loop(0, n_chunks, do_chunk, 0)

    return body(F0, F1, F2, indices_list)


# speedup vs baseline: 1.2954x; 1.2954x over previous
import functools

import jax
import jax.numpy as jnp
from jax import lax
from jax.experimental import pallas as pl
from jax.experimental.pallas import tpu as pltpu
from jax.experimental.pallas import tpu_sc as plsc

_LANES = 16
_CHUNK = 512
_SUB = 128


def kernel(F0, F1, F2, indices_list):
    n = indices_list.shape[1]
    r = F0.shape[1]
    info = plsc.get_sparse_core_info()
    num_workers = info.num_cores * info.num_subcores
    n_per_w = n // num_workers
    n_chunks = n_per_w // _CHUNK

    mesh = plsc.VectorSubcoreMesh(core_axis_name="c", subcore_axis_name="s")

    @functools.partial(
        pl.kernel,
        out_type=jax.ShapeDtypeStruct((n,), jnp.float32),
        mesh=mesh,
        scratch_types=[
            pltpu.VMEM((_CHUNK,), jnp.int32),
            pltpu.VMEM((_CHUNK,), jnp.int32),
            pltpu.VMEM((_CHUNK,), jnp.int32),
            pltpu.VMEM((_CHUNK, r), jnp.float32),
            pltpu.VMEM((_CHUNK, r), jnp.float32),
            pltpu.VMEM((_CHUNK, r), jnp.float32),
            pltpu.VMEM((_CHUNK,), jnp.float32),
            pltpu.SemaphoreType.DMA,
        ],
        compiler_params=pltpu.CompilerParams(
            needs_layout_passes=False, use_tc_tiling_on_sc=False
        ),
    )
    def body(f0_hbm, f1_hbm, f2_hbm, i0_hbm, i1_hbm, i2_hbm, out_hbm,
             i0, i1, i2, g0, g1, g2, ov, sem):
        wid = lax.axis_index("s") * info.num_cores + lax.axis_index("c")
        wbase = wid * n_per_w
        iota = lax.iota(jnp.int32, _LANES)

        def do_chunk(c, carry):
            base = wbase + c * _CHUNK
            pltpu.sync_copy(i0_hbm.at[pl.ds(base, _CHUNK)], i0)
            pltpu.sync_copy(i1_hbm.at[pl.ds(base, _CHUNK)], i1)
            pltpu.sync_copy(i2_hbm.at[pl.ds(base, _CHUNK)], i2)
            copies = []
            for j in range(_CHUNK // _SUB):
                s = pl.ds(j * _SUB, _SUB)
                copies.append(pltpu.async_copy(f0_hbm.at[i0.at[s]], g0.at[s], sem))
                copies.append(pltpu.async_copy(f1_hbm.at[i1.at[s]], g1.at[s], sem))
                copies.append(pltpu.async_copy(f2_hbm.at[i2.at[s]], g2.at[s], sem))
            for cp in copies:
                cp.wait()

            def do_group(g, gcarry):
                rows = g * _LANES + iota
                acc = jnp.zeros((_LANES,), jnp.float32)
                for rr in range(r):
                    col = jnp.full((_LANES,), rr, jnp.int32)
                    v0 = plsc.load_gather(g0, [rows, col])
                    v1 = plsc.load_gather(g1, [rows, col])
                    v2 = plsc.load_gather(g2, [rows, col])
                    acc = acc + v0 * v1 * v2
                ov[pl.ds(g * _LANES, _LANES)] = acc
                return gcarry

            lax.fori_loop(0, _CHUNK // _LANES, do_group, 0)
            pltpu.sync_copy(ov, out_hbm.at[pl.ds(base, _CHUNK)])
            return carry

        lax.fori_loop(0, n_chunks, do_chunk, 0)

    return body(F0, F1, F2, indices_list[0], indices_list[1], indices_list[2])


# speedup vs baseline: 2.2078x; 1.7043x over previous
import functools

import jax
import jax.numpy as jnp
from jax import lax
from jax.experimental import pallas as pl
from jax.experimental.pallas import tpu as pltpu
from jax.experimental.pallas import tpu_sc as plsc

_LANES = 16
_CHUNK = 512
_SUB = 128


def kernel(F0, F1, F2, indices_list):
    n = indices_list.shape[1]
    r = F0.shape[1]
    info = plsc.get_sparse_core_info()
    num_workers = info.num_cores * info.num_subcores
    n_per_w = n // num_workers
    n_chunks = n_per_w // _CHUNK

    mesh = plsc.VectorSubcoreMesh(core_axis_name="c", subcore_axis_name="s")

    @functools.partial(
        pl.kernel,
        out_type=jax.ShapeDtypeStruct((n,), jnp.float32),
        mesh=mesh,
        scratch_types=[
            pltpu.VMEM((_CHUNK,), jnp.int32),
            pltpu.VMEM((_CHUNK,), jnp.int32),
            pltpu.VMEM((_CHUNK,), jnp.int32),
            pltpu.VMEM((_CHUNK, r), jnp.float32),
            pltpu.VMEM((_CHUNK, r), jnp.float32),
            pltpu.VMEM((_CHUNK, r), jnp.float32),
            pltpu.VMEM((_CHUNK,), jnp.float32),
            pltpu.SemaphoreType.DMA,
        ],
        compiler_params=pltpu.CompilerParams(
            needs_layout_passes=False, use_tc_tiling_on_sc=False
        ),
    )
    def body(f0_hbm, f1_hbm, f2_hbm, i0_hbm, i1_hbm, i2_hbm, out_hbm,
             i0, i1, i2, g0, g1, g2, ov, sem):
        wid = lax.axis_index("s") * info.num_cores + lax.axis_index("c")
        wbase = wid * n_per_w
        iota = lax.iota(jnp.int32, _LANES)

        def do_chunk(c, carry):
            base = wbase + c * _CHUNK
            pltpu.sync_copy(i0_hbm.at[pl.ds(base, _CHUNK)], i0)
            pltpu.sync_copy(i1_hbm.at[pl.ds(base, _CHUNK)], i1)
            pltpu.sync_copy(i2_hbm.at[pl.ds(base, _CHUNK)], i2)
            copies = []
            for j in range(_CHUNK // _SUB):
                s = pl.ds(j * _SUB, _SUB)
                copies.append(pltpu.async_copy(f0_hbm.at[i0.at[s]], g0.at[s], sem))
                copies.append(pltpu.async_copy(f1_hbm.at[i1.at[s]], g1.at[s], sem))
                copies.append(pltpu.async_copy(f2_hbm.at[i2.at[s]], g2.at[s], sem))
            for cp in copies:
                cp.wait()

            def do_group(g, gcarry):
                rows = g * _LANES + iota
                acc = jnp.zeros((_LANES,), jnp.float32)
                for rr in range(r):
                    col = (iota + rr) & (r - 1)
                    v0 = plsc.load_gather(g0, [rows, col])
                    v1 = plsc.load_gather(g1, [rows, col])
                    v2 = plsc.load_gather(g2, [rows, col])
                    acc = acc + v0 * v1 * v2
                ov[pl.ds(g * _LANES, _LANES)] = acc
                return gcarry

            lax.fori_loop(0, _CHUNK // _LANES, do_group, 0)
            pltpu.sync_copy(ov, out_hbm.at[pl.ds(base, _CHUNK)])
            return carry

        lax.fori_loop(0, n_chunks, do_chunk, 0)

    return body(F0, F1, F2, indices_list[0], indices_list[1], indices_list[2])
